# baseline (speedup 1.0000x reference)
import functools
import jax
import jax.numpy as jnp
from jax import lax
from jax.experimental import pallas as pl
from jax.experimental.pallas import tpu as pltpu
from jax.experimental.pallas import tpu_sc as plsc

_NB_TF = 4
_N_SEQ_TF = 2
_B, _C, _H, _W = 128, 3, 224, 224
_NC, _NS = 2, 16
_NW = _NC * _NS
_IPW = _B // _NW
_NPIX = _C * _H * _W
_NJ = _W // 16


def _sc_body(x_hbm, ctab_hbm, out_hbm, buf_in, buf_out, ctbuf, redbuf):
    wid = lax.axis_index("s") * _NC + lax.axis_index("c")
    npix = jnp.float32(_NPIX)
    zero16 = jnp.zeros((16,), jnp.float32)
    iota16 = lax.iota(jnp.int32, 16)

    def allsum(v):
        for sh in (8, 4, 2, 1):
            redbuf[...] = v
            rot = plsc.load_gather(redbuf, [(iota16 + sh) & 15])
            v = v + rot
        return v

    def plane_sum(r, accs):
        a, b = accs
        for j in range(_NJ):
            v = buf_in[r, pl.ds(16 * j, 16)]
            if j % 2 == 0:
                a = a + v
            else:
                b = b + v
        return (a, b)

    for k in range(_IPW):
        img = wid * _IPW + k
        pltpu.sync_copy(ctab_hbm.at[img], ctbuf)
        a0v = ctbuf[0]
        g0v = ctbuf[1]
        lo0v = ctbuf[2]
        hi0v = ctbuf[3]
        a1v = ctbuf[4]
        g1v = ctbuf[5]
        lo1v = ctbuf[6]
        hi1v = ctbuf[7]
        flipb = ctbuf[8] > jnp.float32(0.5)

        s0v = zero16
        for c in range(_C):
            pltpu.sync_copy(x_hbm.at[img, c], buf_in)
            accs = lax.fori_loop(0, _H, plane_sum, (zero16, zero16))
            s0v = s0v + accs[0] + accs[1]
        b0v = g0v * (allsum(s0v) / npix)

        def plane_ysum(r, accs):
            a, b = accs
            for j in range(_NJ):
                v = buf_in[r, pl.ds(16 * j, 16)]
                y = jnp.minimum(jnp.maximum(v * a0v + b0v, lo0v), hi0v)
                if j % 2 == 0:
                    a = a + y
                else:
                    b = b + y
            return (a, b)

        s1v = zero16
        for c in range(_C):
            pltpu.sync_copy(x_hbm.at[img, c], buf_in)
            accs = lax.fori_loop(0, _H, plane_ysum, (zero16, zero16))
            s1v = s1v + accs[0] + accs[1]
        b1v = g1v * (allsum(s1v) / npix)

        def plane_out(r, carry):
            rowv = jnp.full((16,), r, jnp.int32)
            for j in range(_NJ):
                fwd = iota16 + (16 * j)
                bwd = jnp.full((16,), _W - 1 - 16 * j, jnp.int32) - iota16
                colv = jnp.where(flipb, bwd, fwd)
                v = plsc.load_gather(buf_in, [rowv, colv])
                y = jnp.minimum(jnp.maximum(v * a0v + b0v, lo0v), hi0v)
                z = jnp.minimum(jnp.maximum(y * a1v + b1v, lo1v), hi1v)
                buf_out[r, pl.ds(16 * j, 16)] = z
            return carry

        for c in range(_C):
            pltpu.sync_copy(x_hbm.at[img, c], buf_in)
            lax.fori_loop(0, _H, plane_out, 0)
            pltpu.sync_copy(buf_out, out_hbm.at[img, c])


def kernel(x, prob, mag, temp):
    batch = x.shape[0]
    temp_d = lax.stop_gradient(temp)
    mag_d = lax.stop_gradient(mag)
    distrib = jax.nn.softmax(prob * temp_d, axis=0)
    logits = jnp.log(distrib + 1e-12)
    skey = jax.random.key(42)
    samples = jax.random.categorical(
        skey, jnp.broadcast_to(logits, (batch, _NB_TF)), axis=-1,
        shape=(_N_SEQ_TF, batch)
    ).astype(jnp.int32)
    s0, s1 = samples[0], samples[1]

    f = jnp.float32(0.5) + mag_d / jnp.float32(1.0)
    one = jnp.float32(1.0)
    zero = jnp.float32(0.0)
    inf = jnp.float32(jnp.inf)

    def coeffs(s):
        c = s >= 2
        a = jnp.where(c, f, one)
        g = jnp.where(s == 3, one - f, zero)
        lo = jnp.where(c, zero, -inf)
        hi = jnp.where(c, one, inf)
        return a, g, lo, hi

    a0, g0, lo0, hi0 = coeffs(s0)
    a1, g1, lo1, hi1 = coeffs(s1)
    flip = ((s0 == 1) != (s1 == 1)).astype(jnp.float32)
    pad = jnp.zeros_like(flip)
    ctab = jnp.stack(
        [a0, g0, lo0, hi0, a1, g1, lo1, hi1, flip, pad, pad, pad], axis=1)
    ctab = jnp.broadcast_to(ctab[:, :, None], (batch, 12, 16))

    mesh = plsc.VectorSubcoreMesh(core_axis_name="c", subcore_axis_name="s")
    sc = pl.kernel(
        _sc_body,
        out_type=jax.ShapeDtypeStruct(x.shape, x.dtype),
        mesh=mesh,
        scratch_types=[
            pltpu.VMEM((_H, _W), jnp.float32),
            pltpu.VMEM((_H, _W), jnp.float32),
            pltpu.VMEM((12, 16), jnp.float32),
            pltpu.VMEM((16,), jnp.float32),
        ],
        compiler_params=pltpu.CompilerParams(needs_layout_passes=False),
    )
    return sc(x, ctab)


# speedup vs baseline: 1.7520x; 1.7520x over previous
import functools
import jax
import jax.numpy as jnp
from jax import lax
from jax.experimental import pallas as pl
from jax.experimental.pallas import tpu as pltpu
from jax.experimental.pallas import tpu_sc as plsc

_NB_TF = 4
_N_SEQ_TF = 2
_B, _C, _H, _W = 128, 3, 224, 224
_NC, _NS = 2, 16
_NW = _NC * _NS
_IPW = _B // _NW
_NPIX = _C * _H * _W
_NJ = _W // 16


def _sc_body(x_hbm, ctab_hbm, out_hbm, buf_in, buf_out, ctbuf, redbuf):
    wid = lax.axis_index("s") * _NC + lax.axis_index("c")
    npix = jnp.float32(_NPIX)
    zero16 = jnp.zeros((16,), jnp.float32)
    iota16 = lax.iota(jnp.int32, 16)

    def allsum(v):
        for sh in (8, 4, 2, 1):
            redbuf[...] = v
            rot = plsc.load_gather(redbuf, [(iota16 + sh) & 15])
            v = v + rot
        return v

    def plane_sum(r, accs):
        a, b = accs
        for j in range(_NJ):
            v = buf_in[r, pl.ds(16 * j, 16)]
            if j % 2 == 0:
                a = a + v
            else:
                b = b + v
        return (a, b)

    for k in range(_IPW):
        img = wid * _IPW + k
        pltpu.sync_copy(ctab_hbm.at[img], ctbuf)
        a0v = ctbuf[0]
        g0v = ctbuf[1]
        lo0v = ctbuf[2]
        hi0v = ctbuf[3]
        a1v = ctbuf[4]
        g1v = ctbuf[5]
        lo1v = ctbuf[6]
        hi1v = ctbuf[7]
        flipb = ctbuf[8] > jnp.float32(0.5)

        s0v = zero16
        for c in range(_C):
            pltpu.sync_copy(x_hbm.at[img, c], buf_in)
            accs = lax.fori_loop(0, _H, plane_sum, (zero16, zero16))
            s0v = s0v + accs[0] + accs[1]
        b0v = g0v * (allsum(s0v) / npix)

        def plane_ysum(r, accs):
            a, b = accs
            for j in range(_NJ):
                v = buf_in[r, pl.ds(16 * j, 16)]
                y = jnp.minimum(jnp.maximum(v * a0v + b0v, lo0v), hi0v)
                if j % 2 == 0:
                    a = a + y
                else:
                    b = b + y
            return (a, b)

        s1v = zero16
        for c in range(_C):
            pltpu.sync_copy(x_hbm.at[img, c], buf_in)
            accs = lax.fori_loop(0, _H, plane_ysum, (zero16, zero16))
            s1v = s1v + accs[0] + accs[1]
        b1v = g1v * (allsum(s1v) / npix)

        def plane_out(r, carry):
            for j in range(_NJ):
                vf = buf_in[r, pl.ds(16 * j, 16)]
                vb = buf_in[r, pl.ds(_W - 16 * (j + 1), 16)]
                v = jnp.where(flipb, lax.rev(vb, dimensions=(0,)), vf)
                y = jnp.minimum(jnp.maximum(v * a0v + b0v, lo0v), hi0v)
                z = jnp.minimum(jnp.maximum(y * a1v + b1v, lo1v), hi1v)
                buf_out[r, pl.ds(16 * j, 16)] = z
            return carry

        for c in range(_C):
            pltpu.sync_copy(x_hbm.at[img, c], buf_in)
            lax.fori_loop(0, _H, plane_out, 0)
            pltpu.sync_copy(buf_out, out_hbm.at[img, c])


def kernel(x, prob, mag, temp):
    batch = x.shape[0]
    temp_d = lax.stop_gradient(temp)
    mag_d = lax.stop_gradient(mag)
    distrib = jax.nn.softmax(prob * temp_d, axis=0)
    logits = jnp.log(distrib + 1e-12)
    skey = jax.random.key(42)
    samples = jax.random.categorical(
        skey, jnp.broadcast_to(logits, (batch, _NB_TF)), axis=-1,
        shape=(_N_SEQ_TF, batch)
    ).astype(jnp.int32)
    s0, s1 = samples[0], samples[1]

    f = jnp.float32(0.5) + mag_d / jnp.float32(1.0)
    one = jnp.float32(1.0)
    zero = jnp.float32(0.0)
    inf = jnp.float32(jnp.inf)

    def coeffs(s):
        c = s >= 2
        a = jnp.where(c, f, one)
        g = jnp.where(s == 3, one - f, zero)
        lo = jnp.where(c, zero, -inf)
        hi = jnp.where(c, one, inf)
        return a, g, lo, hi

    a0, g0, lo0, hi0 = coeffs(s0)
    a1, g1, lo1, hi1 = coeffs(s1)
    flip = ((s0 == 1) != (s1 == 1)).astype(jnp.float32)
    pad = jnp.zeros_like(flip)
    ctab = jnp.stack(
        [a0, g0, lo0, hi0, a1, g1, lo1, hi1, flip, pad, pad, pad], axis=1)
    ctab = jnp.broadcast_to(ctab[:, :, None], (batch, 12, 16))

    mesh = plsc.VectorSubcoreMesh(core_axis_name="c", subcore_axis_name="s")
    sc = pl.kernel(
        _sc_body,
        out_type=jax.ShapeDtypeStruct(x.shape, x.dtype),
        mesh=mesh,
        scratch_types=[
            pltpu.VMEM((_H, _W), jnp.float32),
            pltpu.VMEM((_H, _W), jnp.float32),
            pltpu.VMEM((12, 16), jnp.float32),
            pltpu.VMEM((16,), jnp.float32),
        ],
        compiler_params=pltpu.CompilerParams(needs_layout_passes=False),
    )
    return sc(x, ctab)


# speedup vs baseline: 1.9037x; 1.0866x over previous
import functools
import jax
import jax.numpy as jnp
from jax import lax
from jax.experimental import pallas as pl
from jax.experimental.pallas import tpu as pltpu
from jax.experimental.pallas import tpu_sc as plsc

_NB_TF = 4
_N_SEQ_TF = 2
_B, _C, _H, _W = 128, 3, 224, 224
_NC, _NS = 2, 16
_NW = _NC * _NS
_IPW = _B // _NW
_NPIX = _C * _H * _W
_NJ = _W // 16
_NBF = 4
_CR = _H // 2
_NT = _C * 2


def _sc_body(x_hbm, ctab_hbm, out_hbm, bufs, ctbuf, redbuf, sin, sout):
    wid = lax.axis_index("s") * _NC + lax.axis_index("c")
    npix = jnp.float32(_NPIX)
    zero16 = jnp.zeros((16,), jnp.float32)
    iota16 = lax.iota(jnp.int32, 16)

    def allsum(v):
        for sh in (8, 4, 2, 1):
            redbuf[...] = v
            rot = plsc.load_gather(redbuf, [(iota16 + sh) & 15])
            v = v + rot
        return v

    def chunk_in(img, t):
        j = t % _NBF
        return pltpu.async_copy(
            x_hbm.at[img, t // 2, pl.ds((t % 2) * _CR, _CR)], bufs.at[j],
            sin.at[j])

    def chunk_out(img, t):
        j = t % _NBF
        return pltpu.async_copy(
            bufs.at[j], out_hbm.at[img, t // 2, pl.ds((t % 2) * _CR, _CR)],
            sout.at[j])

    def reduce_pass(img, row_fn):
        accv = zero16
        hs = [chunk_in(img, t) for t in range(3)]
        for t in range(_NT):
            j = t % _NBF
            hs[t].wait()
            if t + 3 < _NT:
                hs.append(chunk_in(img, t + 3))
            accs = lax.fori_loop(0, _CR, functools.partial(row_fn, j),
                                 (zero16, zero16))
            accv = accv + accs[0] + accs[1]
        return allsum(accv) / npix

    for k in range(_IPW):
        img = wid * _IPW + k
        pltpu.sync_copy(ctab_hbm.at[img], ctbuf)
        a0v = ctbuf[0]
        g0v = ctbuf[1]
        lo0v = ctbuf[2]
        hi0v = ctbuf[3]
        a1v = ctbuf[4]
        g1v = ctbuf[5]
        lo1v = ctbuf[6]
        hi1v = ctbuf[7]
        flipb = ctbuf[8] > jnp.float32(0.5)

        def row_sum(j, r, accs):
            a, b = accs
            for q in range(_NJ):
                v = bufs[j, r, pl.ds(16 * q, 16)]
                if q % 2 == 0:
                    a = a + v
                else:
                    b = b + v
            return (a, b)

        b0v = g0v * reduce_pass(img, row_sum)

        def row_ysum(j, r, accs):
            a, b = accs
            for q in range(_NJ):
                v = bufs[j, r, pl.ds(16 * q, 16)]
                y = jnp.minimum(jnp.maximum(v * a0v + b0v, lo0v), hi0v)
                if q % 2 == 0:
                    a = a + y
                else:
                    b = b + y
            return (a, b)

        b1v = g1v * reduce_pass(img, row_ysum)

        def row_out(j, r, carry):
            for q in range(_NJ // 2):
                qm = _NJ - 1 - q
                vf = bufs[j, r, pl.ds(16 * q, 16)]
                vm = bufs[j, r, pl.ds(16 * qm, 16)]
                va = jnp.where(flipb, lax.rev(vm, dimensions=(0,)), vf)
                vb = jnp.where(flipb, lax.rev(vf, dimensions=(0,)), vm)
                ya = jnp.minimum(jnp.maximum(va * a0v + b0v, lo0v), hi0v)
                yb = jnp.minimum(jnp.maximum(vb * a0v + b0v, lo0v), hi0v)
                za = jnp.minimum(jnp.maximum(ya * a1v + b1v, lo1v), hi1v)
                zb = jnp.minimum(jnp.maximum(yb * a1v + b1v, lo1v), hi1v)
                bufs[j, r, pl.ds(16 * q, 16)] = za
                bufs[j, r, pl.ds(16 * qm, 16)] = zb
            return carry

        hs = [chunk_in(img, t) for t in range(3)]
        outs = {}
        for t in range(_NT):
            j = t % _NBF
            hs[t].wait()
            if t + 3 < _NT:
                if t + 3 >= _NBF:
                    outs[t + 3 - _NBF].wait()
                hs.append(chunk_in(img, t + 3))
            lax.fori_loop(0, _CR, functools.partial(row_out, j), 0)
            outs[t] = chunk_out(img, t)
        for t in range(_NT - _NBF, _NT):
            outs[t].wait()


def kernel(x, prob, mag, temp):
    batch = x.shape[0]
    temp_d = lax.stop_gradient(temp)
    mag_d = lax.stop_gradient(mag)
    distrib = jax.nn.softmax(prob * temp_d, axis=0)
    logits = jnp.log(distrib + 1e-12)
    skey = jax.random.key(42)
    samples = jax.random.categorical(
        skey, jnp.broadcast_to(logits, (batch, _NB_TF)), axis=-1,
        shape=(_N_SEQ_TF, batch)
    ).astype(jnp.int32)
    s0, s1 = samples[0], samples[1]

    f = jnp.float32(0.5) + mag_d / jnp.float32(1.0)
    one = jnp.float32(1.0)
    zero = jnp.float32(0.0)
    inf = jnp.float32(jnp.inf)

    def coeffs(s):
        c = s >= 2
        a = jnp.where(c, f, one)
        g = jnp.where(s == 3, one - f, zero)
        lo = jnp.where(c, zero, -inf)
        hi = jnp.where(c, one, inf)
        return a, g, lo, hi

    a0, g0, lo0, hi0 = coeffs(s0)
    a1, g1, lo1, hi1 = coeffs(s1)
    flip = ((s0 == 1) != (s1 == 1)).astype(jnp.float32)
    pad = jnp.zeros_like(flip)
    ctab = jnp.stack(
        [a0, g0, lo0, hi0, a1, g1, lo1, hi1, flip, pad, pad, pad], axis=1)
    ctab = jnp.broadcast_to(ctab[:, :, None], (batch, 12, 16))

    mesh = plsc.VectorSubcoreMesh(core_axis_name="c", subcore_axis_name="s")
    sc = pl.kernel(
        _sc_body,
        out_type=jax.ShapeDtypeStruct(x.shape, x.dtype),
        mesh=mesh,
        scratch_types=[
            pltpu.VMEM((_NBF, _CR, _W), jnp.float32),
            pltpu.VMEM((12, 16), jnp.float32),
            pltpu.VMEM((16,), jnp.float32),
            pltpu.SemaphoreType.DMA((_NBF,)),
            pltpu.SemaphoreType.DMA((_NBF,)),
        ],
        compiler_params=pltpu.CompilerParams(needs_layout_passes=False),
    )
    return sc(x, ctab)
